# baseline (speedup 1.0000x reference)
import functools

import numpy as np

import jax
import jax.numpy as jnp
from jax import lax
from jax.experimental import pallas as pl
from jax.experimental.pallas import tpu as pltpu
from jax.experimental.pallas import tpu_sc as plsc

_NUM_CORES = 2
_NUM_SUBCORES = 16
_NUM_WORKERS = _NUM_CORES * _NUM_SUBCORES
_LANES = 16
_WIDE = 128
_NP_IOTA = np.arange(_LANES)


def _make_sc_lookup(batch: int, fields: int, vocab_wide: int, dim: int):
    bw = batch // _NUM_WORKERS
    assert bw * _NUM_WORKERS == batch
    half = bw // 2
    hgroups = half // _LANES

    mesh = plsc.VectorSubcoreMesh(
        core_axis_name="c", subcore_axis_name="s",
        num_cores=_NUM_CORES, num_subcores=_NUM_SUBCORES)

    @functools.partial(
        pl.kernel,
        mesh=mesh,
        compiler_params=pltpu.CompilerParams(needs_layout_passes=False),
        out_type=jax.ShapeDtypeStruct((fields, dim, batch), jnp.float32),
        scratch_types=[
            pltpu.VMEM((fields * bw,), jnp.int32),
            pltpu.VMEM((fields * bw,), jnp.int32),
            pltpu.VMEM((half, _WIDE), jnp.float32),
            pltpu.VMEM((half, _WIDE), jnp.float32),
            pltpu.VMEM((dim, bw), jnp.float32),
            pltpu.SemaphoreType.DMA,
            pltpu.SemaphoreType.DMA,
        ],
    )
    def lookup(gidx_hbm, qoff_hbm, table_hbm, out_hbm,
               gidx_v, qoff_v, rows0, rows1, out_t, sem0, sem1):
        wid = lax.axis_index("s") * _NUM_CORES + lax.axis_index("c")
        b0 = wid * bw
        iota = lax.iota(jnp.int32, _LANES)

        def load_idx(f, carry):
            pltpu.sync_copy(gidx_hbm.at[pl.ds(f * batch + b0, bw)],
                            gidx_v.at[pl.ds(f * bw, bw)])
            pltpu.sync_copy(qoff_hbm.at[pl.ds(f * batch + b0, bw)],
                            qoff_v.at[pl.ds(f * bw, bw)])
            return carry
        lax.fori_loop(0, fields, load_idx, 0)

        def fire(t, buf, sem):
            pltpu.async_copy(
                table_hbm.at[gidx_v.at[pl.ds(t * half, half)]], buf, sem)

        def drain(buf, sem):
            pltpu.make_async_copy(
                table_hbm.at[gidx_v.at[pl.ds(0, half)]], buf, sem).wait()

        def extract(t, buf, off):
            @plsc.parallel_loop(0, hgroups, 1, unroll=2)
            def group_body(g):
                rvec = g * _LANES + iota
                qv = qoff_v[pl.ds(t * half + g * _LANES, _LANES)]
                for c in range(dim):
                    cv = (iota + c) & (dim - 1)
                    x = plsc.load_gather(buf, [rvec, qv + cv])
                    s = 1.0 / (1.0 + jnp.exp(-x))
                    plsc.store_scatter(out_t, [cv, off + rvec], s)

        fire(0, rows0, sem0)

        def field_body(f, carry):
            drain(rows0, sem0)
            fire(2 * f + 1, rows1, sem1)
            extract(2 * f, rows0, 0)
            drain(rows1, sem1)

            @pl.when(f < fields - 1)
            def _():
                fire(2 * f + 2, rows0, sem0)

            extract(2 * f + 1, rows1, half)
            pltpu.sync_copy(out_t, out_hbm.at[f, :, pl.ds(b0, bw)])
            return carry

        lax.fori_loop(0, fields, field_body, 0)

    return lookup


_VCOLS = 8192
_SUB = _VCOLS // 4


def _tc_widen(table_t, vocab: int, dim: int):
    rpg = _WIDE // dim
    out_rows = _VCOLS // rpg
    grid = pl.cdiv(vocab, _VCOLS)

    def tbody(x_ref, o_ref):
        x = x_ref[...]
        o_ref[...] = jnp.concatenate(
            [x[:, s * _SUB:(s + 1) * _SUB].T for s in range(rpg)], axis=1)

    return pl.pallas_call(
        tbody,
        grid=(grid,),
        in_specs=[pl.BlockSpec((dim, _VCOLS), lambda i: (0, i))],
        out_specs=pl.BlockSpec((out_rows, _WIDE), lambda i: (i, 0)),
        out_shape=jax.ShapeDtypeStruct((grid * out_rows, _WIDE), jnp.float32),
    )(table_t)


def kernel(features, embedding_table):
    b, f = features.shape
    v, d = embedding_table.shape
    idx = features.T.reshape(f * b)
    lv, ls = _VCOLS.bit_length() - 1, _SUB.bit_length() - 1
    gidx = ((idx >> lv) << ls) + (idx & (_SUB - 1))
    qoff = ((idx >> ls) & (_VCOLS // _SUB - 1)) * d
    table_wide = _tc_widen(embedding_table.T, v, d)
    lookup = _make_sc_lookup(b, f, table_wide.shape[0], d)
    out = lookup(gidx, qoff, table_wide)
    return out.transpose(2, 0, 1)


# speedup vs baseline: 1.0074x; 1.0074x over previous
import functools

import jax
import jax.numpy as jnp
from jax import lax
from jax.experimental import pallas as pl
from jax.experimental.pallas import tpu as pltpu
from jax.experimental.pallas import tpu_sc as plsc

_NUM_CORES = 2
_NUM_SUBCORES = 16
_NUM_WORKERS = _NUM_CORES * _NUM_SUBCORES
_LANES = 16
_WIDE = 128


def _make_sc_lookup(batch: int, fields: int, vocab_wide: int, dim: int):
    bw = batch // _NUM_WORKERS
    assert bw * _NUM_WORKERS == batch
    groups = bw // _LANES

    mesh = plsc.VectorSubcoreMesh(
        core_axis_name="c", subcore_axis_name="s",
        num_cores=_NUM_CORES, num_subcores=_NUM_SUBCORES)

    @functools.partial(
        pl.kernel,
        mesh=mesh,
        compiler_params=pltpu.CompilerParams(needs_layout_passes=False),
        out_type=jax.ShapeDtypeStruct((fields, dim, batch), jnp.float32),
        scratch_types=[
            pltpu.VMEM((bw,), jnp.int32),
            pltpu.VMEM((bw,), jnp.int32),
            pltpu.VMEM((bw, _WIDE), jnp.float32),
            pltpu.VMEM((dim, bw), jnp.float32),
            pltpu.SemaphoreType.DMA,
        ],
    )
    def lookup(gidx_hbm, qoff_hbm, table_hbm, out_hbm,
               gidx_v, qoff_v, rows_v, out_t, sem):
        wid = lax.axis_index("s") * _NUM_CORES + lax.axis_index("c")
        b0 = wid * bw
        iota = lax.iota(jnp.int32, _LANES)

        def field_body(f, carry):
            base = f * batch + b0
            pltpu.sync_copy(gidx_hbm.at[pl.ds(base, bw)], gidx_v)
            pltpu.sync_copy(qoff_hbm.at[pl.ds(base, bw)], qoff_v)
            pltpu.async_copy(table_hbm.at[gidx_v], rows_v, sem).wait()

            @plsc.parallel_loop(0, groups, 1, unroll=2)
            def group_body(g):
                rvec = g * _LANES + iota
                qv = qoff_v[pl.ds(g * _LANES, _LANES)]
                for c in range(dim):
                    cv = (iota + c) & (dim - 1)
                    x = plsc.load_gather(rows_v, [rvec, qv + cv])
                    s = 1.0 / (1.0 + jnp.exp(-x))
                    plsc.store_scatter(out_t, [cv, rvec], s)
            pltpu.sync_copy(out_t, out_hbm.at[f, :, pl.ds(b0, bw)])
            return carry

        lax.fori_loop(0, fields, field_body, 0)

    return lookup


_VCOLS = 8192
_SUB = _VCOLS // 4


def _tc_widen(table_t, vocab: int, dim: int):
    rpg = _WIDE // dim
    out_rows = _VCOLS // rpg
    grid = pl.cdiv(vocab, _VCOLS)

    def tbody(x_ref, o_ref):
        x = x_ref[...]
        o_ref[...] = jnp.concatenate(
            [x[:, s * _SUB:(s + 1) * _SUB].T for s in range(rpg)], axis=1)

    return pl.pallas_call(
        tbody,
        grid=(grid,),
        in_specs=[pl.BlockSpec((dim, _VCOLS), lambda i: (0, i))],
        out_specs=pl.BlockSpec((out_rows, _WIDE), lambda i: (i, 0)),
        out_shape=jax.ShapeDtypeStruct((grid * out_rows, _WIDE), jnp.float32),
    )(table_t)


def kernel(features, embedding_table):
    b, f = features.shape
    v, d = embedding_table.shape
    idx = features.T.reshape(f * b)
    lv, ls = _VCOLS.bit_length() - 1, _SUB.bit_length() - 1
    gidx = ((idx >> lv) << ls) + (idx & (_SUB - 1))
    qoff = ((idx >> ls) & (_VCOLS // _SUB - 1)) * d
    table_wide = _tc_widen(embedding_table.T, v, d)
    lookup = _make_sc_lookup(b, f, table_wide.shape[0], d)
    out = lookup(gidx, qoff, table_wide)
    return out.transpose(2, 0, 1)
